# baseline (speedup 1.0000x reference)
import functools
import jax
import jax.numpy as jnp
from jax import lax
from jax.experimental import pallas as pl
from jax.experimental.pallas import tpu as pltpu
from jax.experimental.pallas import tpu_sc as plsc

N = 10000
E = 320000
D = 128
DA = 8
NC = 2
NS = 16
NW = NC * NS
EPW = E // NW
CH = 80
NCH = EPW // CH
NZT = 10
RPT = N // NZT
BN = 400
GRID = N // BN


def _sc_scatter_body(table_hbm, src_hbm, dst_hbm, zeros_hbm, out_hbm,
                     sidx0, sidx1, sidx2, sidx3,
                     didx0, didx1, didx2, didx3,
                     rows0, rows1, rows2, rows3,
                     semS, semD, semG, semW, acc_sh):
    c = lax.axis_index("c")
    s = lax.axis_index("s")
    wid = c * NS + s
    @pl.when(s < NZT)
    def _():
        pltpu.sync_copy(zeros_hbm.at[pl.ds(s * RPT, RPT)],
                        acc_sh.at[pl.ds(s * RPT, RPT)])
    plsc.subcore_barrier()
    base = wid * EPW
    sidx = [sidx0, sidx1, sidx2, sidx3]
    didx = [didx0, didx1, didx2, didx3]
    rows = [rows0, rows1, rows2, rows3]

    def issue_idx(p, j):
        off = pl.multiple_of(base + j * CH, 8)
        pltpu.async_copy(src_hbm.at[pl.ds(off, CH)], sidx[p], semS.at[p])
        pltpu.async_copy(dst_hbm.at[pl.ds(off, CH)], didx[p], semD.at[p])

    def wait_s(p):
        pltpu.make_async_copy(src_hbm.at[pl.ds(0, CH)], sidx[p],
                              semS.at[p]).wait()

    def wait_d(p):
        pltpu.make_async_copy(dst_hbm.at[pl.ds(0, CH)], didx[p],
                              semD.at[p]).wait()

    def issue_gather(p):
        pltpu.async_copy(table_hbm.at[sidx[p]], rows[p], semG.at[p])

    def wait_gather(p):
        pltpu.make_async_copy(table_hbm.at[pl.ds(0, CH)], rows[p],
                              semG.at[p]).wait()

    def issue_scatter(p):
        pltpu.async_copy(rows[p], acc_sh.at[didx[p]], semW.at[p], add=True)

    def wait_scatter(p):
        pltpu.make_async_copy(table_hbm.at[pl.ds(0, CH)], rows[p],
                              semW.at[p]).wait()

    issue_idx(0, 0)
    issue_idx(1, 1)
    issue_idx(2, 2)
    wait_s(0)
    issue_gather(0)

    def body(j, carry):
        p = jnp.remainder(j, 4)
        _ = p

        def at(j_off, fn):
            for q in range(4):
                pl.when(jnp.remainder(j + j_off, 4) == q)(lambda q=q: fn(q))

        at(0, wait_gather)
        at(0, wait_d)
        at(0, issue_scatter)

        @pl.when(j > 0)
        def _():
            at(-1, wait_scatter)

        @pl.when(j < NCH - 3)
        def _():
            at(3, lambda q: issue_idx(q, j + 3))

        @pl.when(j < NCH - 1)
        def _():
            at(1, wait_s)
            at(1, issue_gather)
        return carry

    lax.fori_loop(0, NCH, body, 0)
    for q in range(4):
        pl.when(jnp.remainder(NCH - 1, 4) == q)(lambda q=q: wait_scatter(q))
    plsc.subcore_barrier()

    @pl.when(s < NZT)
    def _():
        pltpu.sync_copy(acc_sh.at[pl.ds(s * RPT, RPT)],
                        out_hbm.at[c, pl.ds(s * RPT, RPT)])


FLN = N * DA
CHE = 2000
NECH = EPW // CHE
NGRP = CHE // 16
DF = 6


def _sc_pre_body(ef_hbm, dst_hbm, zeros_hbm, out_hbm,
                 dat_v, dsti_v, accp_v):
    c = lax.axis_index("c")
    s = lax.axis_index("s")
    wid = c * NS + s
    pltpu.sync_copy(zeros_hbm, accp_v)
    lane = lax.iota(jnp.int32, 16)
    in_b = jnp.logical_or(jnp.logical_and(lane >= DF, lane < 2 * DF),
                          lane == 13)
    off_c = jnp.where(lane < DF, lane,
                      jnp.where(lane < 2 * DF, lane - DF, DF))
    mask = lane < 14
    is_cnt = lane >= 2 * DF
    base6 = wid * EPW * DF
    based = wid * EPW

    def chunk(i, carry):
        off6 = pl.multiple_of(base6 + i * (CHE * DF), 8)
        offd = pl.multiple_of(based + i * CHE, 8)
        pltpu.sync_copy(ef_hbm.at[pl.ds(off6, CHE * DF)],
                        dat_v.at[pl.ds(0, CHE * DF)])
        pltpu.sync_copy(dst_hbm.at[pl.ds(offd, CHE)],
                        dsti_v.at[pl.ds(0, CHE)])

        def group(g, carry2):
            dvec = dsti_v[pl.ds(g * 16, 16)]
            for k in range(8):
                d0 = dvec[2 * k]
                d1 = dvec[2 * k + 1]
                dv = dat_v[pl.ds((g * 16 + 2 * k) * DF, 16)]
                dv = jnp.where(is_cnt, 1.0, dv)
                iv = jnp.where(in_b, d1, d0) * DA + off_c
                plsc.addupdate_scatter(accp_v, [iv], dv, mask=mask)
            return carry2

        lax.fori_loop(0, NGRP, group, 0)
        return carry

    lax.fori_loop(0, NECH, chunk, 0)
    obase = pl.multiple_of(wid * FLN, 8)
    pltpu.sync_copy(accp_v, out_hbm.at[pl.ds(obase, FLN)])


@functools.cache
def _sc_kernels():
    mesh = plsc.VectorSubcoreMesh(core_axis_name="c", subcore_axis_name="s",
                                  num_cores=NC, num_subcores=NS)
    sc_scatter = pl.kernel(
        _sc_scatter_body,
        out_type=jax.ShapeDtypeStruct((NC, N, D), jnp.float32),
        mesh=mesh,
        scratch_types=[
            pltpu.VMEM((CH,), jnp.int32),
            pltpu.VMEM((CH,), jnp.int32),
            pltpu.VMEM((CH,), jnp.int32),
            pltpu.VMEM((CH,), jnp.int32),
            pltpu.VMEM((CH,), jnp.int32),
            pltpu.VMEM((CH,), jnp.int32),
            pltpu.VMEM((CH,), jnp.int32),
            pltpu.VMEM((CH,), jnp.int32),
            pltpu.VMEM((CH, D), jnp.float32),
            pltpu.VMEM((CH, D), jnp.float32),
            pltpu.VMEM((CH, D), jnp.float32),
            pltpu.VMEM((CH, D), jnp.float32),
            pltpu.SemaphoreType.DMA((4,)),
            pltpu.SemaphoreType.DMA((4,)),
            pltpu.SemaphoreType.DMA((4,)),
            pltpu.SemaphoreType.DMA((4,)),
            pltpu.VMEM_SHARED((N, D), jnp.float32),
        ],
    )
    sc_pre = pl.kernel(
        _sc_pre_body,
        out_type=jax.ShapeDtypeStruct((NW * FLN,), jnp.float32),
        mesh=mesh,
        compiler_params=pltpu.CompilerParams(needs_layout_passes=False),
        scratch_types=[
            pltpu.VMEM((CHE * DF + 16,), jnp.float32),
            pltpu.VMEM((CHE + 16,), jnp.int32),
            pltpu.VMEM((FLN,), jnp.float32),
        ],
    )
    return sc_scatter, sc_pre


SEGB = BN * DA


def _segsum_body(p_ref, o_ref):
    o_ref[...] = jnp.sum(p_ref[...], axis=0, keepdims=True)


_segsum = pl.pallas_call(
    _segsum_body,
    grid=(GRID,),
    in_specs=[pl.BlockSpec((NW, SEGB), lambda i: (0, i))],
    out_specs=pl.BlockSpec((1, SEGB), lambda i: (0, i)),
    out_shape=jax.ShapeDtypeStruct((1, FLN), jnp.float32),
)


def _mm0_body(x_ref, w_ref, b_ref, o_ref):
    o_ref[...] = lax.dot_general(
        x_ref[...], w_ref[...], (((1,), (1,)), ((), ())),
        preferred_element_type=jnp.float32) + b_ref[...]


_mm0 = pl.pallas_call(
    _mm0_body,
    grid=(GRID,),
    in_specs=[
        pl.BlockSpec((BN, D), lambda i: (i, 0)),
        pl.BlockSpec((D, D), lambda i: (0, 0)),
        pl.BlockSpec((1, D), lambda i: (0, 0)),
    ],
    out_specs=pl.BlockSpec((BN, D), lambda i: (i, 0)),
    out_shape=jax.ShapeDtypeStruct((N, D), jnp.float32),
)


def _fused_body(a0_ref, a1_ref, s_ref, we_ref, sc_ref, sh_ref,
                w_ref, b_ref, o_ref):
    edge_term = lax.dot_general(
        s_ref[...], we_ref[...], (((1,), (1,)), ((), ())),
        preferred_element_type=jnp.float32)
    agg = a0_ref[...] + a1_ref[...] + edge_term
    x = jnp.maximum(agg, 0.0) * sc_ref[...] + sh_ref[...]
    o_ref[...] = lax.dot_general(
        x, w_ref[...], (((1,), (1,)), ((), ())),
        preferred_element_type=jnp.float32) + b_ref[...]


_fused = pl.pallas_call(
    _fused_body,
    grid=(GRID,),
    in_specs=[
        pl.BlockSpec((BN, D), lambda i: (i, 0)),
        pl.BlockSpec((BN, D), lambda i: (i, 0)),
        pl.BlockSpec((BN, DA), lambda i: (i, 0)),
        pl.BlockSpec((D, DA), lambda i: (0, 0)),
        pl.BlockSpec((1, D), lambda i: (0, 0)),
        pl.BlockSpec((1, D), lambda i: (0, 0)),
        pl.BlockSpec((D, D), lambda i: (0, 0)),
        pl.BlockSpec((1, D), lambda i: (0, 0)),
    ],
    out_specs=pl.BlockSpec((BN, D), lambda i: (i, 0)),
    out_shape=jax.ShapeDtypeStruct((N, D), jnp.float32),
)


def _final_body(a0_ref, a1_ref, s_ref, we_ref, sc_ref, sh_ref,
                o_ref, acc_ref):
    i = pl.program_id(0)

    @pl.when(i == 0)
    def _():
        acc_ref[...] = jnp.zeros_like(acc_ref)

    edge_term = lax.dot_general(
        s_ref[...], we_ref[...], (((1,), (1,)), ((), ())),
        preferred_element_type=jnp.float32)
    agg = a0_ref[...] + a1_ref[...] + edge_term
    x = jnp.maximum(agg, 0.0) * sc_ref[...] + sh_ref[...]
    acc_ref[...] += jnp.sum(x, axis=0, keepdims=True)

    @pl.when(i == pl.num_programs(0) - 1)
    def _():
        o_ref[...] = acc_ref[...] * (1.0 / N)


_final = pl.pallas_call(
    _final_body,
    grid=(GRID,),
    in_specs=[
        pl.BlockSpec((BN, D), lambda i: (i, 0)),
        pl.BlockSpec((BN, D), lambda i: (i, 0)),
        pl.BlockSpec((BN, DA), lambda i: (i, 0)),
        pl.BlockSpec((D, DA), lambda i: (0, 0)),
        pl.BlockSpec((1, D), lambda i: (0, 0)),
        pl.BlockSpec((1, D), lambda i: (0, 0)),
    ],
    out_specs=pl.BlockSpec((1, D), lambda i: (0, 0)),
    out_shape=jax.ShapeDtypeStruct((1, D), jnp.float32),
    scratch_shapes=[pltpu.VMEM((1, D), jnp.float32)],
)


@jax.jit
def _run(nodeFeatures, edgeIndex, edgeFeatures, W, b, We, be, gamma, beta,
         running_mean, running_var):
    src = edgeIndex[:, 0]
    dst = edgeIndex[:, 1]
    ef_flat = edgeFeatures.reshape(E * DF)
    we_aug = jnp.concatenate(
        [We, be[:, :, None], jnp.zeros((3, D, 1), jnp.float32)], axis=2)
    scale = gamma * lax.rsqrt(running_var + 1e-5)
    shift = beta - running_mean * scale
    zeros_nd = jnp.zeros((N, D), jnp.float32)
    zeros_fl = jnp.zeros((FLN,), jnp.float32)

    sc_scatter, sc_pre = _sc_kernels()
    parts = sc_pre(ef_flat, dst, zeros_fl).reshape(NW, FLN)
    seg = _segsum(parts).reshape(N, DA)
    table = _mm0(nodeFeatures, W[0], b[0][None])
    for i in range(3):
        acc = sc_scatter(table, src, dst, zeros_nd)
        if i < 2:
            table = _fused(acc[0], acc[1], seg, we_aug[i],
                           scale[i][None], shift[i][None],
                           W[i + 1], b[i + 1][None])
    out = _final(acc[0], acc[1], seg, we_aug[2],
                 scale[2][None], shift[2][None])
    return out[0]


def kernel(nodeFeatures, edgeIndex, edgeFeatures, W, b, We, be, gamma, beta,
           running_mean, running_var):
    return _run(nodeFeatures, edgeIndex, edgeFeatures, W, b, We, be,
                gamma, beta, running_mean, running_var)


# speedup vs baseline: 1.1390x; 1.1390x over previous
import functools
import jax
import jax.numpy as jnp
from jax import lax
from jax.experimental import pallas as pl
from jax.experimental.pallas import tpu as pltpu
from jax.experimental.pallas import tpu_sc as plsc

N = 10000
E = 320000
D = 128
DA = 8
NC = 2
NS = 16
NW = NC * NS
EPW = E // NW
CH = 80
NCH = EPW // CH
NZT = 10
RPT = N // NZT
BN = 400
GRID = N // BN


def _sc_scatter_body(table_hbm, src_hbm, dst_hbm, zeros_hbm, out_hbm,
                     sidx0, sidx1, sidx2, sidx3,
                     didx0, didx1, didx2, didx3,
                     rows0, rows1, rows2, rows3,
                     semS, semD, semG, semW, acc_sh):
    c = lax.axis_index("c")
    s = lax.axis_index("s")
    wid = c * NS + s
    @pl.when(s < NZT)
    def _():
        pltpu.sync_copy(zeros_hbm.at[pl.ds(s * RPT, RPT)],
                        acc_sh.at[pl.ds(s * RPT, RPT)])
    plsc.subcore_barrier()
    base = wid * EPW
    sidx = [sidx0, sidx1, sidx2, sidx3]
    didx = [didx0, didx1, didx2, didx3]
    rows = [rows0, rows1, rows2, rows3]

    def issue_idx(p, j):
        off = pl.multiple_of(base + j * CH, 8)
        pltpu.async_copy(src_hbm.at[pl.ds(off, CH)], sidx[p], semS.at[p])
        pltpu.async_copy(dst_hbm.at[pl.ds(off, CH)], didx[p], semD.at[p])

    def wait_s(p):
        pltpu.make_async_copy(src_hbm.at[pl.ds(0, CH)], sidx[p],
                              semS.at[p]).wait()

    def wait_d(p):
        pltpu.make_async_copy(dst_hbm.at[pl.ds(0, CH)], didx[p],
                              semD.at[p]).wait()

    def issue_gather(p):
        pltpu.async_copy(table_hbm.at[sidx[p]], rows[p], semG.at[p])

    def wait_gather(p):
        pltpu.make_async_copy(table_hbm.at[pl.ds(0, CH)], rows[p],
                              semG.at[p]).wait()

    def issue_scatter(p):
        pltpu.async_copy(rows[p], acc_sh.at[didx[p]], semW.at[p], add=True)

    def wait_scatter(p):
        pltpu.make_async_copy(table_hbm.at[pl.ds(0, CH)], rows[p],
                              semW.at[p]).wait()

    issue_idx(0, 0)
    issue_idx(1, 1)
    issue_idx(2, 2)
    issue_idx(3, 3)
    wait_s(0)
    issue_gather(0)
    NIT = NCH // 4

    def body(i, carry):
        c0 = 4 * i

        @pl.when(i > 0)
        def _():
            wait_scatter(3)
            issue_idx(3, c0 + 3)
        wait_s(1)
        issue_gather(1)
        wait_gather(0)
        wait_d(0)
        issue_scatter(0)
        wait_s(2)
        issue_gather(2)
        wait_gather(1)
        wait_d(1)
        issue_scatter(1)
        wait_s(3)
        issue_gather(3)
        wait_gather(2)
        wait_d(2)
        issue_scatter(2)
        wait_scatter(0)
        issue_idx(0, c0 + 4)
        wait_gather(3)
        wait_d(3)
        issue_scatter(3)

        @pl.when(i < NIT - 1)
        def _():
            wait_scatter(1)
            issue_idx(1, c0 + 5)
            wait_scatter(2)
            issue_idx(2, c0 + 6)
            wait_s(0)
            issue_gather(0)
        return carry

    lax.fori_loop(0, NIT, body, 0)
    wait_scatter(1)
    wait_scatter(2)
    wait_s(0)
    issue_gather(0)
    wait_gather(0)
    wait_d(0)
    issue_scatter(0)
    wait_scatter(0)
    wait_scatter(3)
    plsc.subcore_barrier()

    @pl.when(s < NZT)
    def _():
        pltpu.sync_copy(acc_sh.at[pl.ds(s * RPT, RPT)],
                        out_hbm.at[c, pl.ds(s * RPT, RPT)])


FLN = N * DA
CHE = 2000
NECH = EPW // CHE
NGRP = CHE // 16
DF = 6


def _sc_pre_body(ef_hbm, dst_hbm, zeros_hbm, out_hbm,
                 dat_v, dsti_v, accp_v):
    c = lax.axis_index("c")
    s = lax.axis_index("s")
    wid = c * NS + s
    pltpu.sync_copy(zeros_hbm, accp_v)
    lane = lax.iota(jnp.int32, 16)
    in_b = jnp.logical_or(jnp.logical_and(lane >= DF, lane < 2 * DF),
                          lane == 13)
    off_c = jnp.where(lane < DF, lane,
                      jnp.where(lane < 2 * DF, lane - DF, DF))
    mask = lane < 14
    is_cnt = lane >= 2 * DF
    base6 = wid * EPW * DF
    based = wid * EPW

    def chunk(i, carry):
        off6 = pl.multiple_of(base6 + i * (CHE * DF), 8)
        offd = pl.multiple_of(based + i * CHE, 8)
        pltpu.sync_copy(ef_hbm.at[pl.ds(off6, CHE * DF)],
                        dat_v.at[pl.ds(0, CHE * DF)])
        pltpu.sync_copy(dst_hbm.at[pl.ds(offd, CHE)],
                        dsti_v.at[pl.ds(0, CHE)])

        def group(g, carry2):
            dvec = dsti_v[pl.ds(g * 16, 16)]
            for k in range(8):
                d0 = dvec[2 * k]
                d1 = dvec[2 * k + 1]
                dv = dat_v[pl.ds((g * 16 + 2 * k) * DF, 16)]
                dv = jnp.where(is_cnt, 1.0, dv)
                iv = jnp.where(in_b, d1, d0) * DA + off_c
                plsc.addupdate_scatter(accp_v, [iv], dv, mask=mask)
            return carry2

        lax.fori_loop(0, NGRP, group, 0)
        return carry

    lax.fori_loop(0, NECH, chunk, 0)
    obase = pl.multiple_of(wid * FLN, 8)
    pltpu.sync_copy(accp_v, out_hbm.at[pl.ds(obase, FLN)])


@functools.cache
def _sc_kernels():
    mesh = plsc.VectorSubcoreMesh(core_axis_name="c", subcore_axis_name="s",
                                  num_cores=NC, num_subcores=NS)
    sc_scatter = pl.kernel(
        _sc_scatter_body,
        out_type=jax.ShapeDtypeStruct((NC, N, D), jnp.float32),
        mesh=mesh,
        scratch_types=[
            pltpu.VMEM((CH,), jnp.int32),
            pltpu.VMEM((CH,), jnp.int32),
            pltpu.VMEM((CH,), jnp.int32),
            pltpu.VMEM((CH,), jnp.int32),
            pltpu.VMEM((CH,), jnp.int32),
            pltpu.VMEM((CH,), jnp.int32),
            pltpu.VMEM((CH,), jnp.int32),
            pltpu.VMEM((CH,), jnp.int32),
            pltpu.VMEM((CH, D), jnp.float32),
            pltpu.VMEM((CH, D), jnp.float32),
            pltpu.VMEM((CH, D), jnp.float32),
            pltpu.VMEM((CH, D), jnp.float32),
            pltpu.SemaphoreType.DMA((4,)),
            pltpu.SemaphoreType.DMA((4,)),
            pltpu.SemaphoreType.DMA((4,)),
            pltpu.SemaphoreType.DMA((4,)),
            pltpu.VMEM_SHARED((N, D), jnp.float32),
        ],
    )
    sc_pre = pl.kernel(
        _sc_pre_body,
        out_type=jax.ShapeDtypeStruct((NW * FLN,), jnp.float32),
        mesh=mesh,
        compiler_params=pltpu.CompilerParams(needs_layout_passes=False),
        scratch_types=[
            pltpu.VMEM((CHE * DF + 16,), jnp.float32),
            pltpu.VMEM((CHE + 16,), jnp.int32),
            pltpu.VMEM((FLN,), jnp.float32),
        ],
    )
    return sc_scatter, sc_pre


SEGB = BN * DA


def _segsum_body(p_ref, o_ref):
    o_ref[...] = jnp.sum(p_ref[...], axis=0, keepdims=True)


_segsum = pl.pallas_call(
    _segsum_body,
    grid=(GRID,),
    in_specs=[pl.BlockSpec((NW, SEGB), lambda i: (0, i))],
    out_specs=pl.BlockSpec((1, SEGB), lambda i: (0, i)),
    out_shape=jax.ShapeDtypeStruct((1, FLN), jnp.float32),
)


def _mm0_body(x_ref, w_ref, b_ref, o_ref):
    o_ref[...] = lax.dot_general(
        x_ref[...], w_ref[...], (((1,), (1,)), ((), ())),
        preferred_element_type=jnp.float32) + b_ref[...]


_mm0 = pl.pallas_call(
    _mm0_body,
    grid=(GRID,),
    in_specs=[
        pl.BlockSpec((BN, D), lambda i: (i, 0)),
        pl.BlockSpec((D, D), lambda i: (0, 0)),
        pl.BlockSpec((1, D), lambda i: (0, 0)),
    ],
    out_specs=pl.BlockSpec((BN, D), lambda i: (i, 0)),
    out_shape=jax.ShapeDtypeStruct((N, D), jnp.float32),
)


def _fused_body(a0_ref, a1_ref, s_ref, we_ref, sc_ref, sh_ref,
                w_ref, b_ref, o_ref):
    edge_term = lax.dot_general(
        s_ref[...], we_ref[...], (((1,), (1,)), ((), ())),
        preferred_element_type=jnp.float32)
    agg = a0_ref[...] + a1_ref[...] + edge_term
    x = jnp.maximum(agg, 0.0) * sc_ref[...] + sh_ref[...]
    o_ref[...] = lax.dot_general(
        x, w_ref[...], (((1,), (1,)), ((), ())),
        preferred_element_type=jnp.float32) + b_ref[...]


_fused = pl.pallas_call(
    _fused_body,
    grid=(GRID,),
    in_specs=[
        pl.BlockSpec((BN, D), lambda i: (i, 0)),
        pl.BlockSpec((BN, D), lambda i: (i, 0)),
        pl.BlockSpec((BN, DA), lambda i: (i, 0)),
        pl.BlockSpec((D, DA), lambda i: (0, 0)),
        pl.BlockSpec((1, D), lambda i: (0, 0)),
        pl.BlockSpec((1, D), lambda i: (0, 0)),
        pl.BlockSpec((D, D), lambda i: (0, 0)),
        pl.BlockSpec((1, D), lambda i: (0, 0)),
    ],
    out_specs=pl.BlockSpec((BN, D), lambda i: (i, 0)),
    out_shape=jax.ShapeDtypeStruct((N, D), jnp.float32),
)


def _final_body(a0_ref, a1_ref, s_ref, we_ref, sc_ref, sh_ref,
                o_ref, acc_ref):
    i = pl.program_id(0)

    @pl.when(i == 0)
    def _():
        acc_ref[...] = jnp.zeros_like(acc_ref)

    edge_term = lax.dot_general(
        s_ref[...], we_ref[...], (((1,), (1,)), ((), ())),
        preferred_element_type=jnp.float32)
    agg = a0_ref[...] + a1_ref[...] + edge_term
    x = jnp.maximum(agg, 0.0) * sc_ref[...] + sh_ref[...]
    acc_ref[...] += jnp.sum(x, axis=0, keepdims=True)

    @pl.when(i == pl.num_programs(0) - 1)
    def _():
        o_ref[...] = acc_ref[...] * (1.0 / N)


_final = pl.pallas_call(
    _final_body,
    grid=(GRID,),
    in_specs=[
        pl.BlockSpec((BN, D), lambda i: (i, 0)),
        pl.BlockSpec((BN, D), lambda i: (i, 0)),
        pl.BlockSpec((BN, DA), lambda i: (i, 0)),
        pl.BlockSpec((D, DA), lambda i: (0, 0)),
        pl.BlockSpec((1, D), lambda i: (0, 0)),
        pl.BlockSpec((1, D), lambda i: (0, 0)),
    ],
    out_specs=pl.BlockSpec((1, D), lambda i: (0, 0)),
    out_shape=jax.ShapeDtypeStruct((1, D), jnp.float32),
    scratch_shapes=[pltpu.VMEM((1, D), jnp.float32)],
)


@jax.jit
def _run(nodeFeatures, edgeIndex, edgeFeatures, W, b, We, be, gamma, beta,
         running_mean, running_var):
    src = edgeIndex[:, 0]
    dst = edgeIndex[:, 1]
    ef_flat = edgeFeatures.reshape(E * DF)
    we_aug = jnp.concatenate(
        [We, be[:, :, None], jnp.zeros((3, D, 1), jnp.float32)], axis=2)
    scale = gamma * lax.rsqrt(running_var + 1e-5)
    shift = beta - running_mean * scale
    zeros_nd = jnp.zeros((N, D), jnp.float32)
    zeros_fl = jnp.zeros((FLN,), jnp.float32)

    sc_scatter, sc_pre = _sc_kernels()
    parts = sc_pre(ef_flat, dst, zeros_fl).reshape(NW, FLN)
    seg = _segsum(parts).reshape(N, DA)
    table = _mm0(nodeFeatures, W[0], b[0][None])
    for i in range(3):
        acc = sc_scatter(table, src, dst, zeros_nd)
        if i < 2:
            table = _fused(acc[0], acc[1], seg, we_aug[i],
                           scale[i][None], shift[i][None],
                           W[i + 1], b[i + 1][None])
    out = _final(acc[0], acc[1], seg, we_aug[2],
                 scale[2][None], shift[2][None])
    return out[0]


def kernel(nodeFeatures, edgeIndex, edgeFeatures, W, b, We, be, gamma, beta,
           running_mean, running_var):
    return _run(nodeFeatures, edgeIndex, edgeFeatures, W, b, We, be,
                gamma, beta, running_mean, running_var)
